# baseline (speedup 1.0000x reference)
import functools

import jax
import jax.numpy as jnp
from jax import lax
from jax.experimental import pallas as pl
from jax.experimental.pallas import tpu as pltpu
from jax.experimental.pallas import tpu_sc as plsc

N = 10000
D = 128
E = 320000

_NC = 2
_NS = 16
_NW = _NC * _NS
_C = 128
_PCH = 16
_NPH = 5
_NCH = _PCH * _NPH
_EPW = _NCH * _C
_EPAD = _NW * _EPW

_RPT_MAIN = 632
_RPT_LAST = N - 15 * _RPT_MAIN


def _bcast_lane(vec16, l):
    dnums = lax.GatherDimensionNumbers(
        offset_dims=(), collapsed_slice_dims=(0,), start_index_map=(0,))
    idx = jnp.full((16, 1), l, jnp.int32)
    return lax.gather(vec16, idx, dnums, slice_sizes=(1,),
                      mode=lax.GatherScatterMode.PROMISE_IN_BOUNDS)


def _sc_scatter(x, packed):
    mesh = plsc.VectorSubcoreMesh(core_axis_name="c", subcore_axis_name="s")

    @functools.partial(
        pl.kernel,
        mesh=mesh,
        out_type=jax.ShapeDtypeStruct((_NC, N, D), jnp.float32),
        scratch_types=[
            pltpu.VMEM((3, _PCH, _C), jnp.int32),
            pltpu.VMEM((3, _PCH, _C), jnp.int32),
            pltpu.VMEM((_C, D), jnp.float32),
            pltpu.VMEM((_C, D), jnp.float32),
            pltpu.VMEM_SHARED((N, D), jnp.float32),
            pltpu.SemaphoreType.DMA,
            pltpu.SemaphoreType.DMA,
            pltpu.SemaphoreType.DMA,
            pltpu.SemaphoreType.DMA,
        ],
    )
    def k(x_hbm, pk_hbm, out_hbm, eb0, eb1, rows0, rows1, acc,
          sem_e0, sem_e1, sem_g0, sem_g1):
        c = lax.axis_index("c")
        s = lax.axis_index("s")
        wid = c * _NS + s
        zero16 = jnp.zeros((16,), jnp.float32)
        ebufs = (eb0, eb1)
        esems = (sem_e0, sem_e1)
        row0 = s * _RPT_MAIN

        def stage(p):
            return pltpu.async_copy(
                pk_hbm.at[wid, :, pl.ds(p * _PCH, _PCH), :],
                ebufs[p % 2], esems[p % 2])

        h0 = stage(0)
        h1 = stage(1)

        def zrow(i, _):
            for j in range(D // 16):
                rows0[i, 16 * j:16 * (j + 1)] = zero16
            return 0
        lax.fori_loop(0, _C, zrow, 0)

        def zero_span(cnt):
            nblk = cnt // _C
            for b in range(nblk):
                pltpu.sync_copy(rows0, acc.at[pl.ds(row0 + b * _C, _C)])
            rem = cnt - nblk * _C
            if rem:
                pltpu.sync_copy(rows0.at[pl.ds(0, rem)],
                                acc.at[pl.ds(row0 + nblk * _C, rem)])

        @pl.when(s < _NS - 1)
        def _():
            zero_span(_RPT_MAIN)

        @pl.when(s == _NS - 1)
        def _():
            zero_span(_RPT_LAST)

        plsc.subcore_barrier()

        def gather(eb, kk, rows_r, sem_g):
            return pltpu.async_copy(x_hbm.at[eb.at[0, kk]], rows_r, sem_g)

        def wait_gather(rows_r):
            pltpu.make_async_copy(
                x_hbm.at[pl.ds(0, _C)], rows_r,
                sem_g0 if rows_r is rows0 else sem_g1).wait()

        def scale(eb, rows_r, kk):
            def group(g, _):
                wv16i = eb[2, kk, pl.ds(g * 16, 16)]
                wv16 = lax.bitcast_convert_type(wv16i, jnp.float32)
                for l in range(16):
                    wv = _bcast_lane(wv16, l)
                    e = g * 16 + l
                    for jj in range(D // 16):
                        v = rows_r[e, 16 * jj:16 * (jj + 1)]
                        rows_r[e, 16 * jj:16 * (jj + 1)] = v * wv
                return 0
            lax.fori_loop(0, _C // 16, group, 0)

        def step(eb, kk):
            gather(eb, kk, rows0, sem_g0).wait()
            scale(eb, rows0, kk)
            pltpu.sync_copy(rows0, acc.at[eb.at[1, kk]], add=True)

        stage_handles = [h0, h1]
        for p in range(_NPH):
            eb = ebufs[p % 2]
            stage_handles[p % 2].wait()

            def chunk(i, _, eb=eb):
                step(eb, i)
                return 0
            lax.fori_loop(0, _PCH, chunk, 0)
            if p + 2 < _NPH:
                stage_handles[p % 2] = stage(p + 2)

        plsc.subcore_barrier()

        @pl.when(s < _NS - 1)
        def _():
            pltpu.sync_copy(acc.at[pl.ds(row0, _RPT_MAIN)],
                            out_hbm.at[c, pl.ds(row0, _RPT_MAIN)])

        @pl.when(s == _NS - 1)
        def _():
            pltpu.sync_copy(acc.at[pl.ds(row0, _RPT_LAST)],
                            out_hbm.at[c, pl.ds(row0, _RPT_LAST)])

    return k(x, packed)


def _tc_finish(p0, p1, W):
    B = 1000

    def body(p0_ref, p1_ref, w_ref, o_ref):
        acc = p0_ref[...] + p1_ref[...]
        o_ref[...] = jnp.maximum(
            jnp.dot(acc, w_ref[...], preferred_element_type=jnp.float32), 0.0)

    return pl.pallas_call(
        body,
        grid=(N // B,),
        in_specs=[
            pl.BlockSpec((B, D), lambda i: (i, 0)),
            pl.BlockSpec((B, D), lambda i: (i, 0)),
            pl.BlockSpec((D, D), lambda i: (0, 0)),
        ],
        out_specs=pl.BlockSpec((B, D), lambda i: (i, 0)),
        out_shape=jax.ShapeDtypeStruct((N, D), jnp.float32),
    )(p0, p1, W)


def kernel(x, edge_index, edge_weight, W):
    src = edge_index[1]
    dst = edge_index[0]
    wbits = lax.bitcast_convert_type(edge_weight, jnp.int32)
    pad = _EPAD - E
    packed = jnp.stack([
        jnp.pad(src, (0, pad)),
        jnp.pad(dst, (0, pad)),
        jnp.pad(wbits, (0, pad)),
    ]).reshape(3, _NW, _NCH, _C).transpose(1, 0, 2, 3)
    partials = _sc_scatter(x, packed)
    return _tc_finish(partials[0], partials[1], W)


# speedup vs baseline: 1.8585x; 1.8585x over previous
import functools

import jax
import jax.numpy as jnp
from jax import lax
from jax.experimental import pallas as pl
from jax.experimental.pallas import tpu as pltpu
from jax.experimental.pallas import tpu_sc as plsc

N = 10000
D = 128
E = 320000

_NC = 2
_NS = 16
_NW = _NC * _NS
_EPW = E // _NW
_C = 128
_FULL = _EPW // _C
_TAIL = _EPW - _FULL * _C

_RPT_MAIN = 632
_RPT_LAST = N - 15 * _RPT_MAIN


def _bcast_lane(vec16, l):
    dnums = lax.GatherDimensionNumbers(
        offset_dims=(), collapsed_slice_dims=(0,), start_index_map=(0,))
    idx = jnp.full((16, 1), l, jnp.int32)
    return lax.gather(vec16, idx, dnums, slice_sizes=(1,),
                      mode=lax.GatherScatterMode.PROMISE_IN_BOUNDS)


def _sc_scatter(x, src, dst, wgt):
    mesh = plsc.VectorSubcoreMesh(core_axis_name="c", subcore_axis_name="s")

    @functools.partial(
        pl.kernel,
        mesh=mesh,
        out_type=jax.ShapeDtypeStruct((_NC, N, D), jnp.float32),
        scratch_types=[
            pltpu.VMEM((_C,), jnp.int32),
            pltpu.VMEM((_C,), jnp.int32),
            pltpu.VMEM((_C,), jnp.float32),
            pltpu.VMEM((_C, D), jnp.float32),
            pltpu.VMEM((_TAIL,), jnp.int32),
            pltpu.VMEM((_TAIL,), jnp.int32),
            pltpu.VMEM((_TAIL,), jnp.float32),
            pltpu.VMEM((_TAIL, D), jnp.float32),
            pltpu.VMEM_SHARED((N, D), jnp.float32),
            pltpu.SemaphoreType.DMA,
        ],
    )
    def k(x_hbm, src_hbm, dst_hbm, wgt_hbm, out_hbm,
          src_v, dst_v, wgt_v, rows_v, src_t, dst_t, wgt_t, rows_t, acc, sem):
        c = lax.axis_index("c")
        s = lax.axis_index("s")
        wid = c * _NS + s
        lane = lax.iota(jnp.int32, 16)
        zero16 = jnp.zeros((16,), jnp.float32)

        def zrow(i, _):
            for j in range(D // 16):
                rows_v[i, 16 * j:16 * (j + 1)] = zero16
            return 0
        lax.fori_loop(0, _C, zrow, 0)
        row0 = s * _RPT_MAIN

        def zero_span(cnt):
            nblk = cnt // _C
            for b in range(nblk):
                pltpu.sync_copy(rows_v, acc.at[pl.ds(row0 + b * _C, _C)])
            rem = cnt - nblk * _C
            if rem:
                pltpu.sync_copy(rows_v.at[pl.ds(0, rem)],
                                acc.at[pl.ds(row0 + nblk * _C, rem)])

        @pl.when(s < _NS - 1)
        def _():
            zero_span(_RPT_MAIN)

        @pl.when(s == _NS - 1)
        def _():
            zero_span(_RPT_LAST)
        plsc.subcore_barrier()

        ebase = wid * _EPW

        def do_chunk(base, cc, src_r, dst_r, wgt_r, rows_r):
            pltpu.sync_copy(src_hbm.at[pl.ds(base, cc)], src_r)
            pltpu.sync_copy(dst_hbm.at[pl.ds(base, cc)], dst_r)
            pltpu.sync_copy(wgt_hbm.at[pl.ds(base, cc)], wgt_r)
            pltpu.async_copy(x_hbm.at[src_r], rows_r, sem).wait()

            def group(g, _):
                wv16 = wgt_r[pl.ds(g * 16, 16)]
                for l in range(16):
                    wv = _bcast_lane(wv16, l)
                    e = g * 16 + l
                    for jj in range(D // 16):
                        v = rows_r[e, 16 * jj:16 * (jj + 1)]
                        rows_r[e, 16 * jj:16 * (jj + 1)] = v * wv
                return 0
            lax.fori_loop(0, cc // 16, group, 0)
            pltpu.sync_copy(rows_r, acc.at[dst_r], add=True)

        def chunk(kk, _):
            do_chunk(ebase + kk * _C, _C, src_v, dst_v, wgt_v, rows_v)
            return 0
        lax.fori_loop(0, _FULL, chunk, 0)
        if _TAIL:
            do_chunk(ebase + _FULL * _C, _TAIL, src_t, dst_t, wgt_t, rows_t)
        plsc.subcore_barrier()

        @pl.when(s < _NS - 1)
        def _():
            pltpu.sync_copy(acc.at[pl.ds(row0, _RPT_MAIN)],
                            out_hbm.at[c, pl.ds(row0, _RPT_MAIN)])

        @pl.when(s == _NS - 1)
        def _():
            pltpu.sync_copy(acc.at[pl.ds(row0, _RPT_LAST)],
                            out_hbm.at[c, pl.ds(row0, _RPT_LAST)])

    return k(x, src, dst, wgt)


def _tc_finish(p0, p1, W):
    B = 1000

    def body(p0_ref, p1_ref, w_ref, o_ref):
        acc = p0_ref[...] + p1_ref[...]
        o_ref[...] = jnp.maximum(
            jnp.dot(acc, w_ref[...], preferred_element_type=jnp.float32), 0.0)

    return pl.pallas_call(
        body,
        grid=(N // B,),
        in_specs=[
            pl.BlockSpec((B, D), lambda i: (i, 0)),
            pl.BlockSpec((B, D), lambda i: (i, 0)),
            pl.BlockSpec((D, D), lambda i: (0, 0)),
        ],
        out_specs=pl.BlockSpec((B, D), lambda i: (i, 0)),
        out_shape=jax.ShapeDtypeStruct((N, D), jnp.float32),
    )(p0, p1, W)


def kernel(x, edge_index, edge_weight, W):
    src = edge_index[1]
    dst = edge_index[0]
    partials = _sc_scatter(x, src, dst, edge_weight)
    return _tc_finish(partials[0], partials[1], W)
